# baseline (speedup 1.0000x reference)
import functools
import math

import jax
import jax.numpy as jnp
from jax import lax
from jax.experimental import pallas as pl
from jax.experimental.pallas import tpu as pltpu
from jax.experimental.pallas import tpu_sc as plsc

NUM_TOKENS = 16384 * 50
B_DIM = 16384
T_DIM = 50
D_MODEL = 64
SCALE = math.sqrt(D_MODEL)

NC, NS, LANES = 2, 16, 16
NW = NC * NS
TOK_PER_W = NUM_TOKENS // NW
CHUNK = 256
SUB = 128
NSUB = CHUNK // SUB
NCHUNK = TOK_PER_W // CHUNK
IDXROWS_W = TOK_PER_W // SUB


def _gather_body(table_hbm, idx_hbm, out_hbm,
                 idx_all, rows0, rows1, gsem0, gsem1, ssem0, ssem1):
    rows = (rows0, rows1)
    gsem = (gsem0, gsem1)
    ssem = (ssem0, ssem1)

    wid = lax.axis_index("s") * NC + lax.axis_index("c")
    base = wid * TOK_PER_W
    idx_row0 = wid * IDXROWS_W

    pltpu.sync_copy(idx_hbm.at[pl.ds(idx_row0, IDXROWS_W)], idx_all)

    def fire_gathers(chunk, buf):
        for j in range(NSUB):
            pltpu.async_copy(
                table_hbm.at[idx_all.at[chunk * NSUB + j]],
                rows[buf].at[pl.ds(j * SUB, SUB)],
                gsem[buf],
            )

    def wait_gathers(buf):
        for j in range(NSUB):
            pltpu.make_async_copy(
                table_hbm.at[idx_all.at[j]],
                rows[buf].at[pl.ds(j * SUB, SUB)],
                gsem[buf],
            ).wait()

    def fire_store(chunk, buf):
        pltpu.async_copy(
            rows[buf], out_hbm.at[pl.ds(base + chunk * CHUNK, CHUNK)],
            ssem[buf],
        )

    def wait_store(buf):
        pltpu.make_async_copy(
            rows[buf], out_hbm.at[pl.ds(base, CHUNK)], ssem[buf],
        ).wait()

    fire_gathers(0, 0)

    @pl.loop(0, NCHUNK, step=2)
    def _pipeline(c):
        for b in range(2):
            cc = c + b
            nb = 1 - b
            @pl.when(cc >= 1)
            def _():
                wait_store(nb)

            nxt = jnp.minimum(cc + 1, NCHUNK - 1)
            fire_gathers(nxt, nb)

            wait_gathers(b)
            fire_store(cc, b)

    wait_store((NCHUNK - 1) % 2)
    wait_gathers(NCHUNK % 2)


_gather_kernel = functools.partial(
    pl.kernel,
    mesh=plsc.VectorSubcoreMesh(core_axis_name="c", subcore_axis_name="s"),
    out_type=jax.ShapeDtypeStruct((NUM_TOKENS, D_MODEL), jnp.float32),
    compiler_params=pltpu.CompilerParams(use_tc_tiling_on_sc=False),
    scratch_types=[
        pltpu.VMEM((IDXROWS_W, SUB), jnp.int32),
        pltpu.VMEM((CHUNK, D_MODEL), jnp.float32),
        pltpu.VMEM((CHUNK, D_MODEL), jnp.float32),
        pltpu.SemaphoreType.DMA,
        pltpu.SemaphoreType.DMA,
        pltpu.SemaphoreType.DMA,
        pltpu.SemaphoreType.DMA,
    ],
)(_gather_body)


TBLK = 256
JBLK = B_DIM // TBLK


def _scale_body(idx_ref, raw_ref, out_ref):
    x = raw_ref[0, 0]
    ya = jnp.swapaxes(x[:, :D_MODEL], 0, 1)
    yb = jnp.swapaxes(x[:, D_MODEL:], 0, 1)
    y = jnp.concatenate([ya, yb], axis=1)
    idx = idx_ref[0, 0]
    s = jnp.where(idx == 0, 0.0, SCALE).astype(jnp.float32)
    out_ref[0] = y * s[None, :]


_scale_kernel = pl.pallas_call(
    _scale_body,
    grid=(T_DIM, JBLK),
    in_specs=[
        pl.BlockSpec((1, 1, TBLK), lambda t, j: (t * JBLK + j, 0, 0)),
        pl.BlockSpec((1, 1, TBLK // 2, 2 * D_MODEL), lambda t, j: (t, j, 0, 0)),
    ],
    out_specs=pl.BlockSpec((1, D_MODEL, TBLK), lambda t, j: (t, 0, j)),
    out_shape=jax.ShapeDtypeStruct((T_DIM, D_MODEL, B_DIM), jnp.float32),
)


def kernel(inputs, table):
    idx_tb = inputs.astype(jnp.int32).T
    idx_perm = (idx_tb.reshape(T_DIM, JBLK, 2, TBLK // 2)
                .transpose(0, 1, 3, 2))
    idx2d = idx_perm.reshape(NUM_TOKENS // SUB, SUB)
    raw = _gather_kernel(table, idx2d)
    raw4 = raw.reshape(T_DIM, JBLK, TBLK // 2, 2 * D_MODEL)
    idx3 = idx_tb.reshape(T_DIM * JBLK, 1, TBLK)
    out_tdb = _scale_kernel(idx3, raw4)
    return out_tdb.transpose(2, 0, 1)


# speedup vs baseline: 1.7353x; 1.7353x over previous
import functools
import math

import jax
import jax.numpy as jnp
from jax import lax
from jax.experimental import pallas as pl
from jax.experimental.pallas import tpu as pltpu
from jax.experimental.pallas import tpu_sc as plsc

B_DIM = 16384
T_DIM = 50
D_MODEL = 64
DW = 128
NUM_TOKENS = B_DIM * T_DIM
VOCAB = 1_000_000
SCALE = math.sqrt(D_MODEL)

NC, NS = 2, 16
NW = NC * NS
TOK_PER_W = NUM_TOKENS // NW
CHUNK = 256
SUB = 128
NSUB = CHUNK // SUB
NCHUNK = TOK_PER_W // CHUNK
IDXROWS_W = TOK_PER_W // SUB

PREP_ROWS = 8000


def _prep_body(tbl_ref, out_ref):
    r0 = pl.program_id(0) * PREP_ROWS
    x = tbl_ref[...] * SCALE
    row = r0 + lax.broadcasted_iota(jnp.int32, x.shape, 0)
    x = jnp.where(row == 0, 0.0, x)
    out_ref[...] = jnp.concatenate(
        [x, jnp.zeros_like(x)], axis=1)


_prep_kernel = pl.pallas_call(
    _prep_body,
    grid=(VOCAB // PREP_ROWS,),
    in_specs=[pl.BlockSpec((PREP_ROWS, D_MODEL), lambda i: (i, 0))],
    out_specs=pl.BlockSpec((PREP_ROWS, DW), lambda i: (i, 0)),
    out_shape=jax.ShapeDtypeStruct((VOCAB, DW), jnp.float32),
)


def _gather_body(table_hbm, idx_hbm, out_hbm,
                 idx_all, rows0, rows1, gsem0, gsem1, ssem0, ssem1):
    rows = (rows0, rows1)
    gsem = (gsem0, gsem1)
    ssem = (ssem0, ssem1)

    wid = lax.axis_index("s") * NC + lax.axis_index("c")
    base = wid * TOK_PER_W
    idx_row0 = wid * IDXROWS_W

    pltpu.sync_copy(idx_hbm.at[pl.ds(idx_row0, IDXROWS_W)], idx_all)

    def fire_gathers(chunk, buf):
        for j in range(NSUB):
            pltpu.async_copy(
                table_hbm.at[idx_all.at[chunk * NSUB + j]],
                rows[buf].at[pl.ds(j * SUB, SUB)],
                gsem[buf],
            )

    def wait_gathers(buf):
        for j in range(NSUB):
            pltpu.make_async_copy(
                table_hbm.at[idx_all.at[j]],
                rows[buf].at[pl.ds(j * SUB, SUB)],
                gsem[buf],
            ).wait()

    def fire_store(chunk, buf):
        pltpu.async_copy(
            rows[buf], out_hbm.at[pl.ds(base + chunk * CHUNK, CHUNK)],
            ssem[buf],
        )

    def wait_store(buf):
        pltpu.make_async_copy(
            rows[buf], out_hbm.at[pl.ds(base, CHUNK)], ssem[buf],
        ).wait()

    fire_gathers(0, 0)

    @pl.loop(0, NCHUNK, step=2)
    def _pipeline(c):
        for b in range(2):
            cc = c + b
            nb = 1 - b
            @pl.when(cc >= 1)
            def _():
                wait_store(nb)

            nxt = jnp.minimum(cc + 1, NCHUNK - 1)
            fire_gathers(nxt, nb)

            wait_gathers(b)
            fire_store(cc, b)

    wait_store((NCHUNK - 1) % 2)
    wait_gathers(NCHUNK % 2)


_gather_kernel = functools.partial(
    pl.kernel,
    mesh=plsc.VectorSubcoreMesh(core_axis_name="c", subcore_axis_name="s"),
    out_type=jax.ShapeDtypeStruct((NUM_TOKENS, DW), jnp.float32),
    compiler_params=pltpu.CompilerParams(use_tc_tiling_on_sc=True),
    scratch_types=[
        pltpu.VMEM((IDXROWS_W, SUB), jnp.int32),
        pltpu.VMEM((CHUNK, DW), jnp.float32),
        pltpu.VMEM((CHUNK, DW), jnp.float32),
        pltpu.SemaphoreType.DMA,
        pltpu.SemaphoreType.DMA,
        pltpu.SemaphoreType.DMA,
        pltpu.SemaphoreType.DMA,
    ],
)(_gather_body)


def kernel(inputs, table):
    table128 = _prep_kernel(table)
    idx2d = inputs.astype(jnp.int32).reshape(NUM_TOKENS // SUB, SUB)
    raw = _gather_kernel(table128, idx2d)
    return raw[:, :D_MODEL].reshape(B_DIM, T_DIM, D_MODEL)
